# baseline (speedup 1.0000x reference)
import functools

import jax
import jax.numpy as jnp
import numpy as np
from jax.experimental import pallas as pl
from jax.experimental.pallas import tpu as pltpu
from jax.experimental.pallas import tpu_sc as plsc

_H = 12
_DH = 64
_EPS = 1e-06
_Z_LOSS_COEF = 0.001
_TOPK = 2


def _qkv_kernel(x_ref, ln_ref, wq_ref, wk_ref, wv_ref, q_ref, k_ref, v_ref):
    x = x_ref[...]
    var = jnp.mean(x * x, axis=1, keepdims=True)
    xn = (ln_ref[...] * (x * jax.lax.rsqrt(var + _EPS))).astype(jnp.bfloat16)
    dims = (((1,), (1,)), ((), ()))
    q_ref[...] = jax.lax.dot_general(xn, wq_ref[...], dims,
                                     preferred_element_type=jnp.float32)
    k_ref[...] = jax.lax.dot_general(xn, wk_ref[...], dims,
                                     preferred_element_type=jnp.float32)
    v_ref[...] = jax.lax.dot_general(xn, wv_ref[...], dims,
                                     preferred_element_type=jnp.float32)


def _attn_kernel(q_ref, k_ref, v_ref, cos_ref, sin_ref, o_ref):
    q = q_ref[0]
    k = k_ref[0]
    v = v_ref[0]
    cos = cos_ref[...]
    sin = sin_ref[...]
    half = _DH // 2
    q_rot = jnp.concatenate([-q[:, half:], q[:, :half]], axis=1)
    k_rot = jnp.concatenate([-k[:, half:], k[:, :half]], axis=1)
    qr = (q * cos + q_rot * sin).astype(jnp.bfloat16)
    kr = (k * cos + k_rot * sin).astype(jnp.bfloat16)
    scores = jax.lax.dot_general(qr, kr, (((1,), (1,)), ((), ())),
                                 preferred_element_type=jnp.float32)
    scores = scores * (1.0 / np.sqrt(_DH).astype(np.float32))
    m = jnp.max(scores, axis=1, keepdims=True)
    p = jnp.exp(scores - m)
    p = (p / jnp.sum(p, axis=1, keepdims=True)).astype(jnp.bfloat16)
    o_ref[0] = jnp.dot(p, v.astype(jnp.bfloat16),
                       preferred_element_type=jnp.float32)


def _post_attn_kernel(a_ref, res_ref, wo_ref, ln_ref, wg_ref,
                      h_ref, hn_ref, ti_ref, tw_ref, zacc_ref):
    a = a_ref[...].astype(jnp.bfloat16)
    h = res_ref[...] + jax.lax.dot_general(
        a, wo_ref[...], (((1,), (1,)), ((), ())),
        preferred_element_type=jnp.float32)
    h_ref[...] = h
    var = jnp.mean(h * h, axis=1, keepdims=True)
    hn = ln_ref[...] * (h * jax.lax.rsqrt(var + _EPS))
    hn_ref[...] = hn.astype(jnp.bfloat16)
    logits = jax.lax.dot_general(hn, wg_ref[...], (((1,), (1,)), ((), ())),
                                 preferred_element_type=jnp.float32)
    e = logits.shape[1]
    lm = jnp.max(logits, axis=1, keepdims=True)
    ex = jnp.exp(logits - lm)
    p = ex / jnp.sum(ex, axis=1, keepdims=True)
    iota = jax.lax.broadcasted_iota(jnp.int32, p.shape, 1)
    m1 = jnp.max(p, axis=1, keepdims=True)
    i1 = jnp.min(jnp.where(p == m1, iota, e), axis=1, keepdims=True)
    sel1 = iota == i1
    p2 = jnp.where(sel1, -1.0, p)
    m2 = jnp.max(p2, axis=1, keepdims=True)
    i2 = jnp.min(jnp.where(p2 == m2, iota, e), axis=1, keepdims=True)
    denom = m1 + m2
    ti_ref[...] = jnp.concatenate([i1, i2], axis=1)
    tw_ref[...] = jnp.concatenate([m1, m2], axis=1) / denom
    z = lm + jnp.log(jnp.sum(ex, axis=1, keepdims=True))
    tile_sum = jnp.sum(z * z, axis=0, keepdims=True)

    @pl.when(pl.program_id(0) == 0)
    def _():
        zacc_ref[...] = jnp.zeros_like(zacc_ref)

    zacc_ref[...] += tile_sum


def _moe_kernel(eid_ref, xs_ref, aw_ref, wg_ref, wu_ref, wd_ref, o_ref):
    xs = xs_ref[...]
    dims = (((1,), (1,)), ((), ()))
    g = jax.lax.dot_general(xs, wg_ref[0], dims,
                            preferred_element_type=jnp.float32)
    u = jax.lax.dot_general(xs, wu_ref[0], dims,
                            preferred_element_type=jnp.float32)
    a = (jax.nn.silu(g) * u).astype(jnp.bfloat16)
    y = jax.lax.dot_general(a, wd_ref[0], dims,
                            preferred_element_type=jnp.float32)
    o_ref[...] = (aw_ref[...] * y).astype(jnp.bfloat16)


def _combine_kernel(h_ref, g1_ref, g2_ref, o_ref):
    o_ref[...] = (h_ref[...] + g1_ref[...].astype(jnp.float32)
                  + g2_ref[...].astype(jnp.float32))


def _sc_gather_rows(data, idx, window=384):
    n, dfull = data.shape
    orig_dtype = data.dtype
    if orig_dtype == jnp.bfloat16:
        data = jax.lax.bitcast_convert_type(
            data.reshape(n, dfull // 2, 2), jnp.int32)
        dpack = dfull // 2
    else:
        dpack = dfull
    rep = dpack // 128
    data = data.reshape(n * rep, 128)
    sub = (idx[:, None] * rep +
           jnp.arange(rep, dtype=jnp.int32)[None, :]).reshape(-1)
    m = sub.shape[0]
    d = 128
    idx2 = sub.reshape(1, m)
    mesh = plsc.VectorSubcoreMesh(core_axis_name="core",
                                  subcore_axis_name="subcore")

    @functools.partial(
        pl.kernel,
        out_type=jax.ShapeDtypeStruct((m, d), data.dtype),
        mesh=mesh)
    def k(x_hbm, i_hbm, o_hbm):
        def body(i_vmem, o_vmem):
            pltpu.sync_copy(x_hbm.at[i_vmem.at[0]], o_vmem)

        pltpu.emit_pipeline(
            body,
            grid=(m // window,),
            in_specs=[pl.BlockSpec((1, window), index_map=lambda i: (0, i))],
            out_specs=[pl.BlockSpec((window, d), index_map=lambda i: (i, 0))],
            core_axis_name=("core", "subcore"),
            dimension_semantics=(pltpu.PARALLEL,),
        )(i_hbm, o_hbm)

    out = k(data, idx2).reshape(-1, dpack)
    if orig_dtype == jnp.bfloat16:
        out = jax.lax.bitcast_convert_type(out, jnp.bfloat16)
    return out.reshape(-1, dfull)


@jax.jit
def kernel(hidden_states, ln1_w, ln2_w, Wq, Wk, Wv, Wo, Wg, We_gate, We_up, We_down):
    B, S, D = hidden_states.shape
    E, FF, _ = We_gate.shape
    T = B * S
    TM = 512
    x = hidden_states.reshape(T, D)

    f32 = jnp.float32
    bf16 = jnp.bfloat16
    ln1 = ln1_w.reshape(1, D)
    ln2 = ln2_w.reshape(1, D)
    Wq = Wq.astype(bf16)
    Wk = Wk.astype(bf16)
    Wv = Wv.astype(bf16)
    Wo = Wo.astype(bf16)
    We_gate = We_gate.astype(bf16)
    We_up = We_up.astype(bf16)
    We_down = We_down.astype(bf16)

    q, k, v = pl.pallas_call(
        _qkv_kernel,
        grid=(T // TM,),
        in_specs=[
            pl.BlockSpec((TM, D), lambda t: (t, 0)),
            pl.BlockSpec((1, D), lambda t: (0, 0)),
            pl.BlockSpec((D, D), lambda t: (0, 0)),
            pl.BlockSpec((D, D), lambda t: (0, 0)),
            pl.BlockSpec((D, D), lambda t: (0, 0)),
        ],
        out_specs=[
            pl.BlockSpec((TM, D), lambda t: (t, 0)),
            pl.BlockSpec((TM, D), lambda t: (t, 0)),
            pl.BlockSpec((TM, D), lambda t: (t, 0)),
        ],
        out_shape=[jax.ShapeDtypeStruct((T, D), f32)] * 3,
    )(x, ln1, Wq, Wk, Wv)

    def to_heads(t):
        return (t.reshape(B, S, _H, _DH).transpose(0, 2, 1, 3)
                .reshape(B * _H, S, _DH))

    qh, kh, vh = to_heads(q), to_heads(k), to_heads(v)

    inv_freq = 1.0 / (10000.0 ** (jnp.arange(0, _DH, 2, dtype=f32) / _DH))
    t_pos = jnp.arange(S, dtype=f32)
    freqs = jnp.outer(t_pos, inv_freq)
    emb = jnp.concatenate([freqs, freqs], axis=-1)
    cos = jnp.cos(emb)
    sin = jnp.sin(emb)

    attn = pl.pallas_call(
        _attn_kernel,
        grid=(B * _H,),
        in_specs=[
            pl.BlockSpec((1, S, _DH), lambda i: (i, 0, 0)),
            pl.BlockSpec((1, S, _DH), lambda i: (i, 0, 0)),
            pl.BlockSpec((1, S, _DH), lambda i: (i, 0, 0)),
            pl.BlockSpec((S, _DH), lambda i: (0, 0)),
            pl.BlockSpec((S, _DH), lambda i: (0, 0)),
        ],
        out_specs=pl.BlockSpec((1, S, _DH), lambda i: (i, 0, 0)),
        out_shape=jax.ShapeDtypeStruct((B * _H, S, _DH), f32),
    )(qh, kh, vh, cos, sin)

    attn_flat = (attn.reshape(B, _H, S, _DH).transpose(0, 2, 1, 3)
                 .reshape(T, D))

    h, hn, ti, tw, zacc = pl.pallas_call(
        _post_attn_kernel,
        grid=(T // TM,),
        in_specs=[
            pl.BlockSpec((TM, D), lambda t: (t, 0)),
            pl.BlockSpec((TM, D), lambda t: (t, 0)),
            pl.BlockSpec((D, D), lambda t: (0, 0)),
            pl.BlockSpec((1, D), lambda t: (0, 0)),
            pl.BlockSpec((E, D), lambda t: (0, 0)),
        ],
        out_specs=[
            pl.BlockSpec((TM, D), lambda t: (t, 0)),
            pl.BlockSpec((TM, D), lambda t: (t, 0)),
            pl.BlockSpec((TM, _TOPK), lambda t: (t, 0)),
            pl.BlockSpec((TM, _TOPK), lambda t: (t, 0)),
            pl.BlockSpec((1, 1), lambda t: (0, 0)),
        ],
        out_shape=[
            jax.ShapeDtypeStruct((T, D), f32),
            jax.ShapeDtypeStruct((T, D), bf16),
            jax.ShapeDtypeStruct((T, _TOPK), jnp.int32),
            jax.ShapeDtypeStruct((T, _TOPK), f32),
            jax.ShapeDtypeStruct((1, 1), f32),
        ],
    )(attn_flat, x, Wo, ln2, Wg)

    aux_loss = _Z_LOSS_COEF * zacc[0, 0] / T

    TB = 256
    A = T * _TOPK
    NP = A + E * TB
    G = NP // TB

    eid = ti.reshape(A)
    wts = tw.reshape(A)
    order = jnp.argsort(eid)
    sorted_eid = eid[order]
    bounds = jnp.searchsorted(sorted_eid, jnp.arange(E + 1, dtype=jnp.int32))
    counts = (bounds[1:] - bounds[:-1]).astype(jnp.int32)
    start = bounds[:-1].astype(jnp.int32)
    pc = ((counts + TB - 1) // TB) * TB
    pend = jnp.cumsum(pc)
    pstart = pend - pc
    j = jnp.arange(A, dtype=jnp.int32)
    dst = pstart[sorted_eid] + (j - start[sorted_eid])
    src_row = jnp.zeros((NP,), dtype=jnp.int32).at[dst].set(order // _TOPK)
    aw = jnp.zeros((NP,), f32).at[dst].set(wts[order]).reshape(NP, 1)
    tile_eid = jnp.clip(
        jnp.searchsorted(pend, jnp.arange(G, dtype=jnp.int32) * TB,
                         side="right"),
        0, E - 1).astype(jnp.int32)
    pos = jnp.zeros((A,), jnp.int32).at[order].set(dst).reshape(T, _TOPK)

    xs = _sc_gather_rows(hn, src_row)

    ys = pl.pallas_call(
        _moe_kernel,
        grid_spec=pltpu.PrefetchScalarGridSpec(
            num_scalar_prefetch=1,
            grid=(G,),
            in_specs=[
                pl.BlockSpec((TB, D), lambda g, eid_ref: (g, 0)),
                pl.BlockSpec((TB, 1), lambda g, eid_ref: (g, 0)),
                pl.BlockSpec((1, FF, D), lambda g, eid_ref: (eid_ref[g], 0, 0)),
                pl.BlockSpec((1, FF, D), lambda g, eid_ref: (eid_ref[g], 0, 0)),
                pl.BlockSpec((1, D, FF), lambda g, eid_ref: (eid_ref[g], 0, 0)),
            ],
            out_specs=pl.BlockSpec((TB, D), lambda g, eid_ref: (g, 0)),
        ),
        out_shape=jax.ShapeDtypeStruct((NP, D), bf16),
    )(tile_eid, xs, aw, We_gate, We_up, We_down)

    gpair = _sc_gather_rows(ys, pos.T.reshape(A))

    nt = T // TM
    out = pl.pallas_call(
        _combine_kernel,
        grid=(nt,),
        in_specs=[
            pl.BlockSpec((TM, D), lambda t: (t, 0)),
            pl.BlockSpec((TM, D), lambda t: (t, 0)),
            pl.BlockSpec((TM, D), lambda t: (t + nt, 0)),
        ],
        out_specs=pl.BlockSpec((TM, D), lambda t: (t, 0)),
        out_shape=jax.ShapeDtypeStruct((T, D), f32),
    )(h, gpair, gpair)

    return out.reshape(B, S, D), aux_loss


# speedup vs baseline: 1.0430x; 1.0430x over previous
import functools

import jax
import jax.numpy as jnp
import numpy as np
from jax.experimental import pallas as pl
from jax.experimental.pallas import tpu as pltpu
from jax.experimental.pallas import tpu_sc as plsc

_H = 12
_DH = 64
_EPS = 1e-06
_Z_LOSS_COEF = 0.001
_TOPK = 2


def _qkv_kernel(x_ref, ln_ref, wq_ref, wk_ref, wv_ref, q_ref, k_ref, v_ref):
    x = x_ref[...]
    var = jnp.mean(x * x, axis=1, keepdims=True)
    xn = (ln_ref[...] * (x * jax.lax.rsqrt(var + _EPS))).astype(jnp.bfloat16)
    dims = (((1,), (1,)), ((), ()))
    q_ref[...] = jax.lax.dot_general(xn, wq_ref[...].astype(jnp.bfloat16),
                                     dims, preferred_element_type=jnp.float32)
    k_ref[...] = jax.lax.dot_general(xn, wk_ref[...].astype(jnp.bfloat16),
                                     dims, preferred_element_type=jnp.float32)
    v_ref[...] = jax.lax.dot_general(xn, wv_ref[...].astype(jnp.bfloat16),
                                     dims, preferred_element_type=jnp.float32)


def _attn_kernel(q_ref, k_ref, v_ref, cos_ref, sin_ref, o_ref):
    q = q_ref[0]
    k = k_ref[0]
    v = v_ref[0]
    cos = cos_ref[...]
    sin = sin_ref[...]
    half = _DH // 2
    q_rot = jnp.concatenate([-q[:, half:], q[:, :half]], axis=1)
    k_rot = jnp.concatenate([-k[:, half:], k[:, :half]], axis=1)
    qr = (q * cos + q_rot * sin).astype(jnp.bfloat16)
    kr = (k * cos + k_rot * sin).astype(jnp.bfloat16)
    scores = jax.lax.dot_general(qr, kr, (((1,), (1,)), ((), ())),
                                 preferred_element_type=jnp.float32)
    scores = scores * (1.0 / np.sqrt(_DH).astype(np.float32))
    m = jnp.max(scores, axis=1, keepdims=True)
    p = jnp.exp(scores - m)
    p = (p / jnp.sum(p, axis=1, keepdims=True)).astype(jnp.bfloat16)
    o_ref[0] = jnp.dot(p, v.astype(jnp.bfloat16),
                       preferred_element_type=jnp.float32)


def _post_attn_kernel(a_ref, res_ref, wo_ref, ln_ref, wg_ref,
                      h_ref, hn_ref, ti_ref, tw_ref, zacc_ref):
    a = a_ref[...].astype(jnp.bfloat16)
    h = res_ref[...] + jax.lax.dot_general(
        a, wo_ref[...].astype(jnp.bfloat16), (((1,), (1,)), ((), ())),
        preferred_element_type=jnp.float32)
    h_ref[...] = h
    var = jnp.mean(h * h, axis=1, keepdims=True)
    hn = ln_ref[...] * (h * jax.lax.rsqrt(var + _EPS))
    hn_ref[...] = hn.astype(jnp.bfloat16)
    logits = jax.lax.dot_general(hn, wg_ref[...], (((1,), (1,)), ((), ())),
                                 preferred_element_type=jnp.float32)
    e = logits.shape[1]
    lm = jnp.max(logits, axis=1, keepdims=True)
    ex = jnp.exp(logits - lm)
    p = ex / jnp.sum(ex, axis=1, keepdims=True)
    iota = jax.lax.broadcasted_iota(jnp.int32, p.shape, 1)
    m1 = jnp.max(p, axis=1, keepdims=True)
    i1 = jnp.min(jnp.where(p == m1, iota, e), axis=1, keepdims=True)
    sel1 = iota == i1
    p2 = jnp.where(sel1, -1.0, p)
    m2 = jnp.max(p2, axis=1, keepdims=True)
    i2 = jnp.min(jnp.where(p2 == m2, iota, e), axis=1, keepdims=True)
    denom = m1 + m2
    ti_ref[...] = jnp.concatenate([i1, i2], axis=1)
    tw_ref[...] = jnp.concatenate([m1, m2], axis=1) / denom
    z = lm + jnp.log(jnp.sum(ex, axis=1, keepdims=True))
    tile_sum = jnp.sum(z * z, axis=0, keepdims=True)

    @pl.when(pl.program_id(0) == 0)
    def _():
        zacc_ref[...] = jnp.zeros_like(zacc_ref)

    zacc_ref[...] += tile_sum


def _moe_kernel(eid_ref, xs_ref, aw_ref, wg_ref, wu_ref, wd_ref, o_ref):
    xs = xs_ref[...]
    dims = (((1,), (1,)), ((), ()))
    g = jax.lax.dot_general(xs, wg_ref[0].astype(jnp.bfloat16), dims,
                            preferred_element_type=jnp.float32)
    u = jax.lax.dot_general(xs, wu_ref[0].astype(jnp.bfloat16), dims,
                            preferred_element_type=jnp.float32)
    a = (jax.nn.silu(g) * u).astype(jnp.bfloat16)
    y = jax.lax.dot_general(a, wd_ref[0].astype(jnp.bfloat16), dims,
                            preferred_element_type=jnp.float32)
    o_ref[...] = (aw_ref[...] * y).astype(jnp.bfloat16)


def _combine_kernel(h_ref, g1_ref, g2_ref, o_ref):
    o_ref[...] = (h_ref[...] + g1_ref[...].astype(jnp.float32)
                  + g2_ref[...].astype(jnp.float32))


def _sc_gather_rows(data, idx, window=384):
    n, dfull = data.shape
    orig_dtype = data.dtype
    if orig_dtype == jnp.bfloat16:
        data = jax.lax.bitcast_convert_type(
            data.reshape(n, dfull // 2, 2), jnp.int32)
        dpack = dfull // 2
    else:
        dpack = dfull
    rep = dpack // 128
    data = data.reshape(n * rep, 128)
    sub = (idx[:, None] * rep +
           jnp.arange(rep, dtype=jnp.int32)[None, :]).reshape(-1)
    m = sub.shape[0]
    d = 128
    idx2 = sub.reshape(1, m)
    mesh = plsc.VectorSubcoreMesh(core_axis_name="core",
                                  subcore_axis_name="subcore")

    @functools.partial(
        pl.kernel,
        out_type=jax.ShapeDtypeStruct((m, d), data.dtype),
        mesh=mesh)
    def k(x_hbm, i_hbm, o_hbm):
        def body(i_vmem, o_vmem):
            pltpu.sync_copy(x_hbm.at[i_vmem.at[0]], o_vmem)

        pltpu.emit_pipeline(
            body,
            grid=(m // window,),
            in_specs=[pl.BlockSpec((1, window), index_map=lambda i: (0, i))],
            out_specs=[pl.BlockSpec((window, d), index_map=lambda i: (i, 0))],
            core_axis_name=("core", "subcore"),
            dimension_semantics=(pltpu.PARALLEL,),
        )(i_hbm, o_hbm)

    out = k(data, idx2).reshape(-1, dpack)
    if orig_dtype == jnp.bfloat16:
        out = jax.lax.bitcast_convert_type(out, jnp.bfloat16)
    return out.reshape(-1, dfull)


@jax.jit
def kernel(hidden_states, ln1_w, ln2_w, Wq, Wk, Wv, Wo, Wg, We_gate, We_up, We_down):
    B, S, D = hidden_states.shape
    E, FF, _ = We_gate.shape
    T = B * S
    TM = 512
    x = hidden_states.reshape(T, D)

    f32 = jnp.float32
    bf16 = jnp.bfloat16
    ln1 = ln1_w.reshape(1, D)
    ln2 = ln2_w.reshape(1, D)

    q, k, v = pl.pallas_call(
        _qkv_kernel,
        grid=(T // TM,),
        in_specs=[
            pl.BlockSpec((TM, D), lambda t: (t, 0)),
            pl.BlockSpec((1, D), lambda t: (0, 0)),
            pl.BlockSpec((D, D), lambda t: (0, 0)),
            pl.BlockSpec((D, D), lambda t: (0, 0)),
            pl.BlockSpec((D, D), lambda t: (0, 0)),
        ],
        out_specs=[
            pl.BlockSpec((TM, D), lambda t: (t, 0)),
            pl.BlockSpec((TM, D), lambda t: (t, 0)),
            pl.BlockSpec((TM, D), lambda t: (t, 0)),
        ],
        out_shape=[jax.ShapeDtypeStruct((T, D), f32)] * 3,
    )(x, ln1, Wq, Wk, Wv)

    def to_heads(t):
        return (t.reshape(B, S, _H, _DH).transpose(0, 2, 1, 3)
                .reshape(B * _H, S, _DH))

    qh, kh, vh = to_heads(q), to_heads(k), to_heads(v)

    inv_freq = 1.0 / (10000.0 ** (jnp.arange(0, _DH, 2, dtype=f32) / _DH))
    t_pos = jnp.arange(S, dtype=f32)
    freqs = jnp.outer(t_pos, inv_freq)
    emb = jnp.concatenate([freqs, freqs], axis=-1)
    cos = jnp.cos(emb)
    sin = jnp.sin(emb)

    attn = pl.pallas_call(
        _attn_kernel,
        grid=(B * _H,),
        in_specs=[
            pl.BlockSpec((1, S, _DH), lambda i: (i, 0, 0)),
            pl.BlockSpec((1, S, _DH), lambda i: (i, 0, 0)),
            pl.BlockSpec((1, S, _DH), lambda i: (i, 0, 0)),
            pl.BlockSpec((S, _DH), lambda i: (0, 0)),
            pl.BlockSpec((S, _DH), lambda i: (0, 0)),
        ],
        out_specs=pl.BlockSpec((1, S, _DH), lambda i: (i, 0, 0)),
        out_shape=jax.ShapeDtypeStruct((B * _H, S, _DH), f32),
    )(qh, kh, vh, cos, sin)

    attn_flat = (attn.reshape(B, _H, S, _DH).transpose(0, 2, 1, 3)
                 .reshape(T, D))

    h, hn, ti, tw, zacc = pl.pallas_call(
        _post_attn_kernel,
        grid=(T // TM,),
        in_specs=[
            pl.BlockSpec((TM, D), lambda t: (t, 0)),
            pl.BlockSpec((TM, D), lambda t: (t, 0)),
            pl.BlockSpec((D, D), lambda t: (0, 0)),
            pl.BlockSpec((1, D), lambda t: (0, 0)),
            pl.BlockSpec((E, D), lambda t: (0, 0)),
        ],
        out_specs=[
            pl.BlockSpec((TM, D), lambda t: (t, 0)),
            pl.BlockSpec((TM, D), lambda t: (t, 0)),
            pl.BlockSpec((TM, _TOPK), lambda t: (t, 0)),
            pl.BlockSpec((TM, _TOPK), lambda t: (t, 0)),
            pl.BlockSpec((1, 1), lambda t: (0, 0)),
        ],
        out_shape=[
            jax.ShapeDtypeStruct((T, D), f32),
            jax.ShapeDtypeStruct((T, D), bf16),
            jax.ShapeDtypeStruct((T, _TOPK), jnp.int32),
            jax.ShapeDtypeStruct((T, _TOPK), f32),
            jax.ShapeDtypeStruct((1, 1), f32),
        ],
    )(attn_flat, x, Wo, ln2, Wg)

    aux_loss = _Z_LOSS_COEF * zacc[0, 0] / T

    TB = 256
    A = T * _TOPK
    NP = A + E * TB
    G = NP // TB

    eid = ti.reshape(A)
    wts = tw.reshape(A)
    order = jnp.argsort(eid)
    sorted_eid = eid[order]
    bounds = jnp.searchsorted(sorted_eid, jnp.arange(E + 1, dtype=jnp.int32))
    counts = (bounds[1:] - bounds[:-1]).astype(jnp.int32)
    start = bounds[:-1].astype(jnp.int32)
    pc = ((counts + TB - 1) // TB) * TB
    pend = jnp.cumsum(pc)
    pstart = pend - pc
    j = jnp.arange(A, dtype=jnp.int32)
    dst = pstart[sorted_eid] + (j - start[sorted_eid])
    src_row = jnp.zeros((NP,), dtype=jnp.int32).at[dst].set(order // _TOPK)
    aw = jnp.zeros((NP,), f32).at[dst].set(wts[order]).reshape(NP, 1)
    tile_eid = jnp.clip(
        jnp.searchsorted(pend, jnp.arange(G, dtype=jnp.int32) * TB,
                         side="right"),
        0, E - 1).astype(jnp.int32)
    pos = jnp.zeros((A,), jnp.int32).at[order].set(dst).reshape(T, _TOPK)

    xs = _sc_gather_rows(hn, src_row)

    ys = pl.pallas_call(
        _moe_kernel,
        grid_spec=pltpu.PrefetchScalarGridSpec(
            num_scalar_prefetch=1,
            grid=(G,),
            in_specs=[
                pl.BlockSpec((TB, D), lambda g, eid_ref: (g, 0)),
                pl.BlockSpec((TB, 1), lambda g, eid_ref: (g, 0)),
                pl.BlockSpec((1, FF, D), lambda g, eid_ref: (eid_ref[g], 0, 0)),
                pl.BlockSpec((1, FF, D), lambda g, eid_ref: (eid_ref[g], 0, 0)),
                pl.BlockSpec((1, D, FF), lambda g, eid_ref: (eid_ref[g], 0, 0)),
            ],
            out_specs=pl.BlockSpec((TB, D), lambda g, eid_ref: (g, 0)),
        ),
        out_shape=jax.ShapeDtypeStruct((NP, D), bf16),
    )(tile_eid, xs, aw, We_gate, We_up, We_down)

    gpair = _sc_gather_rows(ys, pos.T.reshape(A))

    nt = T // TM
    out = pl.pallas_call(
        _combine_kernel,
        grid=(nt,),
        in_specs=[
            pl.BlockSpec((TM, D), lambda t: (t, 0)),
            pl.BlockSpec((TM, D), lambda t: (t, 0)),
            pl.BlockSpec((TM, D), lambda t: (t + nt, 0)),
        ],
        out_specs=pl.BlockSpec((TM, D), lambda t: (t, 0)),
        out_shape=jax.ShapeDtypeStruct((T, D), f32),
    )(h, gpair, gpair)

    return out.reshape(B, S, D), aux_loss


# speedup vs baseline: 1.5076x; 1.4455x over previous
import functools

import jax
import jax.numpy as jnp
import numpy as np
from jax.experimental import pallas as pl
from jax.experimental.pallas import tpu as pltpu
from jax.experimental.pallas import tpu_sc as plsc

_H = 12
_DH = 64
_EPS = 1e-06
_Z_LOSS_COEF = 0.001
_TOPK = 2


def _qkv_kernel(x_ref, ln_ref, wq_ref, wk_ref, wv_ref, q_ref, k_ref, v_ref):
    x = x_ref[...]
    var = jnp.mean(x * x, axis=1, keepdims=True)
    xn = (ln_ref[...] * (x * jax.lax.rsqrt(var + _EPS))).astype(jnp.bfloat16)
    dims = (((1,), (1,)), ((), ()))
    q_ref[...] = jax.lax.dot_general(xn, wq_ref[...].astype(jnp.bfloat16),
                                     dims, preferred_element_type=jnp.float32)
    k_ref[...] = jax.lax.dot_general(xn, wk_ref[...].astype(jnp.bfloat16),
                                     dims, preferred_element_type=jnp.float32)
    v_ref[...] = jax.lax.dot_general(xn, wv_ref[...].astype(jnp.bfloat16),
                                     dims, preferred_element_type=jnp.float32)


def _attn_kernel(q_ref, k_ref, v_ref, cos_ref, sin_ref, o_ref):
    q = q_ref[0]
    k = k_ref[0]
    v = v_ref[0]
    cos = cos_ref[...]
    sin = sin_ref[...]
    half = _DH // 2
    q_rot = jnp.concatenate([-q[:, half:], q[:, :half]], axis=1)
    k_rot = jnp.concatenate([-k[:, half:], k[:, :half]], axis=1)
    qr = (q * cos + q_rot * sin).astype(jnp.bfloat16)
    kr = (k * cos + k_rot * sin).astype(jnp.bfloat16)
    scores = jax.lax.dot_general(qr, kr, (((1,), (1,)), ((), ())),
                                 preferred_element_type=jnp.float32)
    scores = scores * (1.0 / np.sqrt(_DH).astype(np.float32))
    m = jnp.max(scores, axis=1, keepdims=True)
    p = jnp.exp(scores - m)
    p = (p / jnp.sum(p, axis=1, keepdims=True)).astype(jnp.bfloat16)
    o_ref[0] = jnp.dot(p, v.astype(jnp.bfloat16),
                       preferred_element_type=jnp.float32)


def _post_attn_kernel(a_ref, res_ref, wo_ref, ln_ref, wg_ref,
                      h_ref, hn_ref, ti_ref, tw_ref, zacc_ref):
    a = a_ref[...].astype(jnp.bfloat16)
    h = res_ref[...] + jax.lax.dot_general(
        a, wo_ref[...].astype(jnp.bfloat16), (((1,), (1,)), ((), ())),
        preferred_element_type=jnp.float32)
    h_ref[...] = h
    var = jnp.mean(h * h, axis=1, keepdims=True)
    hn = ln_ref[...] * (h * jax.lax.rsqrt(var + _EPS))
    hn_ref[...] = hn
    logits = jax.lax.dot_general(hn, wg_ref[...], (((1,), (1,)), ((), ())),
                                 preferred_element_type=jnp.float32)
    e = logits.shape[1]
    lm = jnp.max(logits, axis=1, keepdims=True)
    ex = jnp.exp(logits - lm)
    p = ex / jnp.sum(ex, axis=1, keepdims=True)
    iota = jax.lax.broadcasted_iota(jnp.int32, p.shape, 1)
    m1 = jnp.max(p, axis=1, keepdims=True)
    i1 = jnp.min(jnp.where(p == m1, iota, e), axis=1, keepdims=True)
    sel1 = iota == i1
    p2 = jnp.where(sel1, -1.0, p)
    m2 = jnp.max(p2, axis=1, keepdims=True)
    i2 = jnp.min(jnp.where(p2 == m2, iota, e), axis=1, keepdims=True)
    denom = m1 + m2
    ti_ref[...] = jnp.concatenate([i1, i2], axis=1)
    tw_ref[...] = jnp.concatenate([m1, m2], axis=1) / denom
    z = lm + jnp.log(jnp.sum(ex, axis=1, keepdims=True))
    tile_sum = jnp.sum(z * z, axis=0, keepdims=True)

    @pl.when(pl.program_id(0) == 0)
    def _():
        zacc_ref[...] = jnp.zeros_like(zacc_ref)

    zacc_ref[...] += tile_sum


def _moe_kernel(eid_ref, xs_ref, aw_ref, wg_ref, wu_ref, wd_ref, o_ref):
    xs = xs_ref[...].astype(jnp.bfloat16)
    dims = (((1,), (1,)), ((), ()))
    g = jax.lax.dot_general(xs, wg_ref[0].astype(jnp.bfloat16), dims,
                            preferred_element_type=jnp.float32)
    u = jax.lax.dot_general(xs, wu_ref[0].astype(jnp.bfloat16), dims,
                            preferred_element_type=jnp.float32)
    a = (jax.nn.silu(g) * u).astype(jnp.bfloat16)
    y = jax.lax.dot_general(a, wd_ref[0].astype(jnp.bfloat16), dims,
                            preferred_element_type=jnp.float32)
    o_ref[...] = aw_ref[...] * y


def _combine_kernel(h_ref, g1_ref, g2_ref, o_ref):
    o_ref[...] = h_ref[...] + g1_ref[...] + g2_ref[...]


def _sc_gather_rows(data, idx, window=384):
    n, dfull = data.shape
    orig_dtype = data.dtype
    if orig_dtype == jnp.bfloat16:
        data = jax.lax.bitcast_convert_type(
            data.reshape(n, dfull // 2, 2), jnp.int32)
        dpack = dfull // 2
    else:
        dpack = dfull
    rep = dpack // 128
    data = data.reshape(n * rep, 128)
    sub = (idx[:, None] * rep +
           jnp.arange(rep, dtype=jnp.int32)[None, :]).reshape(-1)
    m = sub.shape[0]
    d = 128
    idx2 = sub.reshape(1, m)
    mesh = plsc.VectorSubcoreMesh(core_axis_name="core",
                                  subcore_axis_name="subcore")

    @functools.partial(
        pl.kernel,
        out_type=jax.ShapeDtypeStruct((m, d), data.dtype),
        mesh=mesh)
    def k(x_hbm, i_hbm, o_hbm):
        def body(i_vmem, o_vmem):
            pltpu.sync_copy(x_hbm.at[i_vmem.at[0]], o_vmem)

        pltpu.emit_pipeline(
            body,
            grid=(m // window,),
            in_specs=[pl.BlockSpec((1, window), index_map=lambda i: (0, i))],
            out_specs=[pl.BlockSpec((window, d), index_map=lambda i: (i, 0))],
            core_axis_name=("core", "subcore"),
            dimension_semantics=(pltpu.PARALLEL,),
        )(i_hbm, o_hbm)

    out = k(data, idx2).reshape(-1, dpack)
    if orig_dtype == jnp.bfloat16:
        out = jax.lax.bitcast_convert_type(out, jnp.bfloat16)
    return out.reshape(-1, dfull)


@jax.jit
def kernel(hidden_states, ln1_w, ln2_w, Wq, Wk, Wv, Wo, Wg, We_gate, We_up, We_down):
    B, S, D = hidden_states.shape
    E, FF, _ = We_gate.shape
    T = B * S
    TM = 512
    x = hidden_states.reshape(T, D)

    f32 = jnp.float32
    bf16 = jnp.bfloat16
    ln1 = ln1_w.reshape(1, D)
    ln2 = ln2_w.reshape(1, D)

    q, k, v = pl.pallas_call(
        _qkv_kernel,
        grid=(T // TM,),
        in_specs=[
            pl.BlockSpec((TM, D), lambda t: (t, 0)),
            pl.BlockSpec((1, D), lambda t: (0, 0)),
            pl.BlockSpec((D, D), lambda t: (0, 0)),
            pl.BlockSpec((D, D), lambda t: (0, 0)),
            pl.BlockSpec((D, D), lambda t: (0, 0)),
        ],
        out_specs=[
            pl.BlockSpec((TM, D), lambda t: (t, 0)),
            pl.BlockSpec((TM, D), lambda t: (t, 0)),
            pl.BlockSpec((TM, D), lambda t: (t, 0)),
        ],
        out_shape=[jax.ShapeDtypeStruct((T, D), f32)] * 3,
    )(x, ln1, Wq, Wk, Wv)

    def to_heads(t):
        return (t.reshape(B, S, _H, _DH).transpose(0, 2, 1, 3)
                .reshape(B * _H, S, _DH))

    qh, kh, vh = to_heads(q), to_heads(k), to_heads(v)

    inv_freq = 1.0 / (10000.0 ** (jnp.arange(0, _DH, 2, dtype=f32) / _DH))
    t_pos = jnp.arange(S, dtype=f32)
    freqs = jnp.outer(t_pos, inv_freq)
    emb = jnp.concatenate([freqs, freqs], axis=-1)
    cos = jnp.cos(emb)
    sin = jnp.sin(emb)

    attn = pl.pallas_call(
        _attn_kernel,
        grid=(B * _H,),
        in_specs=[
            pl.BlockSpec((1, S, _DH), lambda i: (i, 0, 0)),
            pl.BlockSpec((1, S, _DH), lambda i: (i, 0, 0)),
            pl.BlockSpec((1, S, _DH), lambda i: (i, 0, 0)),
            pl.BlockSpec((S, _DH), lambda i: (0, 0)),
            pl.BlockSpec((S, _DH), lambda i: (0, 0)),
        ],
        out_specs=pl.BlockSpec((1, S, _DH), lambda i: (i, 0, 0)),
        out_shape=jax.ShapeDtypeStruct((B * _H, S, _DH), f32),
    )(qh, kh, vh, cos, sin)

    attn_flat = (attn.reshape(B, _H, S, _DH).transpose(0, 2, 1, 3)
                 .reshape(T, D))

    h, hn, ti, tw, zacc = pl.pallas_call(
        _post_attn_kernel,
        grid=(T // TM,),
        in_specs=[
            pl.BlockSpec((TM, D), lambda t: (t, 0)),
            pl.BlockSpec((TM, D), lambda t: (t, 0)),
            pl.BlockSpec((D, D), lambda t: (0, 0)),
            pl.BlockSpec((1, D), lambda t: (0, 0)),
            pl.BlockSpec((E, D), lambda t: (0, 0)),
        ],
        out_specs=[
            pl.BlockSpec((TM, D), lambda t: (t, 0)),
            pl.BlockSpec((TM, D), lambda t: (t, 0)),
            pl.BlockSpec((TM, _TOPK), lambda t: (t, 0)),
            pl.BlockSpec((TM, _TOPK), lambda t: (t, 0)),
            pl.BlockSpec((1, 1), lambda t: (0, 0)),
        ],
        out_shape=[
            jax.ShapeDtypeStruct((T, D), f32),
            jax.ShapeDtypeStruct((T, D), f32),
            jax.ShapeDtypeStruct((T, _TOPK), jnp.int32),
            jax.ShapeDtypeStruct((T, _TOPK), f32),
            jax.ShapeDtypeStruct((1, 1), f32),
        ],
    )(attn_flat, x, Wo, ln2, Wg)

    aux_loss = _Z_LOSS_COEF * zacc[0, 0] / T

    TB = 256
    A = T * _TOPK
    NP = A + E * TB
    G = NP // TB

    eid = ti.reshape(A)
    wts = tw.reshape(A)
    order = jnp.argsort(eid)
    sorted_eid = eid[order]
    bounds = jnp.searchsorted(sorted_eid, jnp.arange(E + 1, dtype=jnp.int32))
    counts = (bounds[1:] - bounds[:-1]).astype(jnp.int32)
    start = bounds[:-1].astype(jnp.int32)
    pc = ((counts + TB - 1) // TB) * TB
    pend = jnp.cumsum(pc)
    pstart = pend - pc
    j = jnp.arange(A, dtype=jnp.int32)
    dst = pstart[sorted_eid] + (j - start[sorted_eid])
    src_row = jnp.zeros((NP,), dtype=jnp.int32).at[dst].set(order // _TOPK)
    aw = jnp.zeros((NP,), f32).at[dst].set(wts[order]).reshape(NP, 1)
    tile_eid = jnp.clip(
        jnp.searchsorted(pend, jnp.arange(G, dtype=jnp.int32) * TB,
                         side="right"),
        0, E - 1).astype(jnp.int32)
    pos = jnp.zeros((A,), jnp.int32).at[order].set(dst).reshape(T, _TOPK)

    xs = _sc_gather_rows(hn, src_row)

    ys = pl.pallas_call(
        _moe_kernel,
        grid_spec=pltpu.PrefetchScalarGridSpec(
            num_scalar_prefetch=1,
            grid=(G,),
            in_specs=[
                pl.BlockSpec((TB, D), lambda g, eid_ref: (g, 0)),
                pl.BlockSpec((TB, 1), lambda g, eid_ref: (g, 0)),
                pl.BlockSpec((1, FF, D), lambda g, eid_ref: (eid_ref[g], 0, 0)),
                pl.BlockSpec((1, FF, D), lambda g, eid_ref: (eid_ref[g], 0, 0)),
                pl.BlockSpec((1, D, FF), lambda g, eid_ref: (eid_ref[g], 0, 0)),
            ],
            out_specs=pl.BlockSpec((TB, D), lambda g, eid_ref: (g, 0)),
        ),
        out_shape=jax.ShapeDtypeStruct((NP, D), f32),
    )(tile_eid, xs, aw, We_gate, We_up, We_down)

    gpair = _sc_gather_rows(ys, pos.T.reshape(A))

    nt = T // TM
    out = pl.pallas_call(
        _combine_kernel,
        grid=(nt,),
        in_specs=[
            pl.BlockSpec((TM, D), lambda t: (t, 0)),
            pl.BlockSpec((TM, D), lambda t: (t, 0)),
            pl.BlockSpec((TM, D), lambda t: (t + nt, 0)),
        ],
        out_specs=pl.BlockSpec((TM, D), lambda t: (t, 0)),
        out_shape=jax.ShapeDtypeStruct((T, D), f32),
    )(h, gpair, gpair)

    return out.reshape(B, S, D), aux_loss


# speedup vs baseline: 1.5570x; 1.0328x over previous
import functools

import jax
import jax.numpy as jnp
import numpy as np
from jax.experimental import pallas as pl
from jax.experimental.pallas import tpu as pltpu
from jax.experimental.pallas import tpu_sc as plsc

_H = 12
_DH = 64
_EPS = 1e-06
_Z_LOSS_COEF = 0.001
_TOPK = 2


def _qkv_kernel(x_ref, ln_ref, wq_ref, wk_ref, wv_ref, q_ref, k_ref, v_ref):
    x = x_ref[...]
    var = jnp.mean(x * x, axis=1, keepdims=True)
    xn = (ln_ref[...] * (x * jax.lax.rsqrt(var + _EPS))).astype(jnp.bfloat16)
    dims = (((1,), (1,)), ((), ()))
    q_ref[...] = jax.lax.dot_general(xn, wq_ref[...].astype(jnp.bfloat16),
                                     dims, preferred_element_type=jnp.float32)
    k_ref[...] = jax.lax.dot_general(xn, wk_ref[...].astype(jnp.bfloat16),
                                     dims, preferred_element_type=jnp.float32)
    v_ref[...] = jax.lax.dot_general(xn, wv_ref[...].astype(jnp.bfloat16),
                                     dims, preferred_element_type=jnp.float32)


def _attn_kernel(q_ref, k_ref, v_ref, cos_ref, sin_ref, o_ref):
    q = q_ref[0]
    k = k_ref[0]
    v = v_ref[0]
    cos = cos_ref[...]
    sin = sin_ref[...]
    half = _DH // 2
    q_rot = jnp.concatenate([-q[:, half:], q[:, :half]], axis=1)
    k_rot = jnp.concatenate([-k[:, half:], k[:, :half]], axis=1)
    qr = ((q * cos + q_rot * sin) * (1.0 / 8.0)).astype(jnp.bfloat16)
    kr = (k * cos + k_rot * sin).astype(jnp.bfloat16)
    scores = jax.lax.dot_general(qr, kr, (((1,), (1,)), ((), ())),
                                 preferred_element_type=jnp.float32)
    m = jnp.max(scores, axis=1, keepdims=True)
    p = jnp.exp(scores - m)
    s = jnp.sum(p, axis=1, keepdims=True)
    o = jnp.dot(p.astype(jnp.bfloat16), v.astype(jnp.bfloat16),
                preferred_element_type=jnp.float32)
    o_ref[0] = o / s


def _post_attn_kernel(a_ref, res_ref, wo_ref, ln_ref, wg_ref,
                      h_ref, hn_ref, ti_ref, tw_ref, zacc_ref):
    a = a_ref[...].astype(jnp.bfloat16)
    h = res_ref[...] + jax.lax.dot_general(
        a, wo_ref[...].astype(jnp.bfloat16), (((1,), (1,)), ((), ())),
        preferred_element_type=jnp.float32)
    h_ref[...] = h
    var = jnp.mean(h * h, axis=1, keepdims=True)
    hn = ln_ref[...] * (h * jax.lax.rsqrt(var + _EPS))
    hn_ref[...] = hn
    logits = jax.lax.dot_general(hn, wg_ref[...], (((1,), (1,)), ((), ())),
                                 preferred_element_type=jnp.float32)
    e = logits.shape[1]
    lm = jnp.max(logits, axis=1, keepdims=True)
    ex = jnp.exp(logits - lm)
    p = ex / jnp.sum(ex, axis=1, keepdims=True)
    iota = jax.lax.broadcasted_iota(jnp.int32, p.shape, 1)
    m1 = jnp.max(p, axis=1, keepdims=True)
    i1 = jnp.min(jnp.where(p == m1, iota, e), axis=1, keepdims=True)
    sel1 = iota == i1
    p2 = jnp.where(sel1, -1.0, p)
    m2 = jnp.max(p2, axis=1, keepdims=True)
    i2 = jnp.min(jnp.where(p2 == m2, iota, e), axis=1, keepdims=True)
    denom = m1 + m2
    ti_ref[...] = jnp.concatenate([i1, i2], axis=1)
    tw_ref[...] = jnp.concatenate([m1, m2], axis=1) / denom
    z = lm + jnp.log(jnp.sum(ex, axis=1, keepdims=True))
    tile_sum = jnp.sum(z * z, axis=0, keepdims=True)

    @pl.when(pl.program_id(0) == 0)
    def _():
        zacc_ref[...] = jnp.zeros_like(zacc_ref)

    zacc_ref[...] += tile_sum


def _moe_kernel(eid_ref, xs_ref, aw_ref, wg_ref, wu_ref, wd_ref, o_ref):
    xs = xs_ref[...].astype(jnp.bfloat16)
    dims = (((1,), (1,)), ((), ()))
    g = jax.lax.dot_general(xs, wg_ref[0].astype(jnp.bfloat16), dims,
                            preferred_element_type=jnp.float32)
    u = jax.lax.dot_general(xs, wu_ref[0].astype(jnp.bfloat16), dims,
                            preferred_element_type=jnp.float32)
    a = (jax.nn.silu(g) * u).astype(jnp.bfloat16)
    y = jax.lax.dot_general(a, wd_ref[0].astype(jnp.bfloat16), dims,
                            preferred_element_type=jnp.float32)
    o_ref[...] = aw_ref[...] * y


def _combine_kernel(h_ref, g1_ref, g2_ref, o_ref):
    o_ref[...] = h_ref[...] + g1_ref[...] + g2_ref[...]


def _sc_gather_rows(data, idx, window=384):
    n, dfull = data.shape
    orig_dtype = data.dtype
    if orig_dtype == jnp.bfloat16:
        data = jax.lax.bitcast_convert_type(
            data.reshape(n, dfull // 2, 2), jnp.int32)
        dpack = dfull // 2
    else:
        dpack = dfull
    rep = dpack // 128
    data = data.reshape(n * rep, 128)
    sub = (idx[:, None] * rep +
           jnp.arange(rep, dtype=jnp.int32)[None, :]).reshape(-1)
    m = sub.shape[0]
    d = 128
    idx2 = sub.reshape(1, m)
    mesh = plsc.VectorSubcoreMesh(core_axis_name="core",
                                  subcore_axis_name="subcore")

    @functools.partial(
        pl.kernel,
        out_type=jax.ShapeDtypeStruct((m, d), data.dtype),
        mesh=mesh)
    def k(x_hbm, i_hbm, o_hbm):
        def body(i_vmem, o_vmem):
            pltpu.sync_copy(x_hbm.at[i_vmem.at[0]], o_vmem)

        pltpu.emit_pipeline(
            body,
            grid=(m // window,),
            in_specs=[pl.BlockSpec((1, window), index_map=lambda i: (0, i))],
            out_specs=[pl.BlockSpec((window, d), index_map=lambda i: (i, 0))],
            core_axis_name=("core", "subcore"),
            dimension_semantics=(pltpu.PARALLEL,),
        )(i_hbm, o_hbm)

    out = k(data, idx2).reshape(-1, dpack)
    if orig_dtype == jnp.bfloat16:
        out = jax.lax.bitcast_convert_type(out, jnp.bfloat16)
    return out.reshape(-1, dfull)


@jax.jit
def kernel(hidden_states, ln1_w, ln2_w, Wq, Wk, Wv, Wo, Wg, We_gate, We_up, We_down):
    B, S, D = hidden_states.shape
    E, FF, _ = We_gate.shape
    T = B * S
    TM = 512
    x = hidden_states.reshape(T, D)

    f32 = jnp.float32
    bf16 = jnp.bfloat16
    ln1 = ln1_w.reshape(1, D)
    ln2 = ln2_w.reshape(1, D)

    q, k, v = pl.pallas_call(
        _qkv_kernel,
        grid=(T // TM,),
        in_specs=[
            pl.BlockSpec((TM, D), lambda t: (t, 0)),
            pl.BlockSpec((1, D), lambda t: (0, 0)),
            pl.BlockSpec((D, D), lambda t: (0, 0)),
            pl.BlockSpec((D, D), lambda t: (0, 0)),
            pl.BlockSpec((D, D), lambda t: (0, 0)),
        ],
        out_specs=[
            pl.BlockSpec((TM, D), lambda t: (t, 0)),
            pl.BlockSpec((TM, D), lambda t: (t, 0)),
            pl.BlockSpec((TM, D), lambda t: (t, 0)),
        ],
        out_shape=[jax.ShapeDtypeStruct((T, D), f32)] * 3,
    )(x, ln1, Wq, Wk, Wv)

    def to_heads(t):
        return (t.reshape(B, S, _H, _DH).transpose(0, 2, 1, 3)
                .reshape(B * _H, S, _DH))

    qh, kh, vh = to_heads(q), to_heads(k), to_heads(v)

    inv_freq = 1.0 / (10000.0 ** (jnp.arange(0, _DH, 2, dtype=f32) / _DH))
    t_pos = jnp.arange(S, dtype=f32)
    freqs = jnp.outer(t_pos, inv_freq)
    emb = jnp.concatenate([freqs, freqs], axis=-1)
    cos = jnp.cos(emb)
    sin = jnp.sin(emb)

    attn = pl.pallas_call(
        _attn_kernel,
        grid=(B * _H,),
        in_specs=[
            pl.BlockSpec((1, S, _DH), lambda i: (i, 0, 0)),
            pl.BlockSpec((1, S, _DH), lambda i: (i, 0, 0)),
            pl.BlockSpec((1, S, _DH), lambda i: (i, 0, 0)),
            pl.BlockSpec((S, _DH), lambda i: (0, 0)),
            pl.BlockSpec((S, _DH), lambda i: (0, 0)),
        ],
        out_specs=pl.BlockSpec((1, S, _DH), lambda i: (i, 0, 0)),
        out_shape=jax.ShapeDtypeStruct((B * _H, S, _DH), f32),
    )(qh, kh, vh, cos, sin)

    attn_flat = (attn.reshape(B, _H, S, _DH).transpose(0, 2, 1, 3)
                 .reshape(T, D))

    h, hn, ti, tw, zacc = pl.pallas_call(
        _post_attn_kernel,
        grid=(T // TM,),
        in_specs=[
            pl.BlockSpec((TM, D), lambda t: (t, 0)),
            pl.BlockSpec((TM, D), lambda t: (t, 0)),
            pl.BlockSpec((D, D), lambda t: (0, 0)),
            pl.BlockSpec((1, D), lambda t: (0, 0)),
            pl.BlockSpec((E, D), lambda t: (0, 0)),
        ],
        out_specs=[
            pl.BlockSpec((TM, D), lambda t: (t, 0)),
            pl.BlockSpec((TM, D), lambda t: (t, 0)),
            pl.BlockSpec((TM, _TOPK), lambda t: (t, 0)),
            pl.BlockSpec((TM, _TOPK), lambda t: (t, 0)),
            pl.BlockSpec((1, 1), lambda t: (0, 0)),
        ],
        out_shape=[
            jax.ShapeDtypeStruct((T, D), f32),
            jax.ShapeDtypeStruct((T, D), f32),
            jax.ShapeDtypeStruct((T, _TOPK), jnp.int32),
            jax.ShapeDtypeStruct((T, _TOPK), f32),
            jax.ShapeDtypeStruct((1, 1), f32),
        ],
    )(attn_flat, x, Wo, ln2, Wg)

    aux_loss = _Z_LOSS_COEF * zacc[0, 0] / T

    TB = 256
    A = T * _TOPK
    NP = A + E * TB
    G = NP // TB

    eid = ti.reshape(A)
    wts = tw.reshape(A)
    order = jnp.argsort(eid)
    sorted_eid = eid[order]
    bounds = jnp.searchsorted(sorted_eid, jnp.arange(E + 1, dtype=jnp.int32))
    counts = (bounds[1:] - bounds[:-1]).astype(jnp.int32)
    start = bounds[:-1].astype(jnp.int32)
    pc = ((counts + TB - 1) // TB) * TB
    pend = jnp.cumsum(pc)
    pstart = pend - pc
    j = jnp.arange(A, dtype=jnp.int32)
    dst = pstart[sorted_eid] + (j - start[sorted_eid])
    src_row = jnp.zeros((NP,), dtype=jnp.int32).at[dst].set(order // _TOPK)
    aw = jnp.zeros((NP,), f32).at[dst].set(wts[order]).reshape(NP, 1)
    tile_eid = jnp.clip(
        jnp.searchsorted(pend, jnp.arange(G, dtype=jnp.int32) * TB,
                         side="right"),
        0, E - 1).astype(jnp.int32)
    pos = jnp.zeros((A,), jnp.int32).at[order].set(dst).reshape(T, _TOPK)

    xs = _sc_gather_rows(hn, src_row)

    ys = pl.pallas_call(
        _moe_kernel,
        grid_spec=pltpu.PrefetchScalarGridSpec(
            num_scalar_prefetch=1,
            grid=(G,),
            in_specs=[
                pl.BlockSpec((TB, D), lambda g, eid_ref: (g, 0)),
                pl.BlockSpec((TB, 1), lambda g, eid_ref: (g, 0)),
                pl.BlockSpec((1, FF, D), lambda g, eid_ref: (eid_ref[g], 0, 0)),
                pl.BlockSpec((1, FF, D), lambda g, eid_ref: (eid_ref[g], 0, 0)),
                pl.BlockSpec((1, D, FF), lambda g, eid_ref: (eid_ref[g], 0, 0)),
            ],
            out_specs=pl.BlockSpec((TB, D), lambda g, eid_ref: (g, 0)),
        ),
        out_shape=jax.ShapeDtypeStruct((NP, D), f32),
    )(tile_eid, xs, aw, We_gate, We_up, We_down)

    gpair = _sc_gather_rows(ys, pos.T.reshape(A))

    nt = T // TM
    out = pl.pallas_call(
        _combine_kernel,
        grid=(nt,),
        in_specs=[
            pl.BlockSpec((TM, D), lambda t: (t, 0)),
            pl.BlockSpec((TM, D), lambda t: (t, 0)),
            pl.BlockSpec((TM, D), lambda t: (t + nt, 0)),
        ],
        out_specs=pl.BlockSpec((TM, D), lambda t: (t, 0)),
        out_shape=jax.ShapeDtypeStruct((T, D), f32),
    )(h, gpair, gpair)

    return out.reshape(B, S, D), aux_loss


# speedup vs baseline: 1.7474x; 1.1223x over previous
import jax
import jax.numpy as jnp
import numpy as np
from jax.experimental import pallas as pl

_H = 12
_DH = 64
_EPS = 1e-06
_Z_LOSS_COEF = 0.001
_TOPK = 2


def _qkv_kernel(x_ref, ln_ref, wq_ref, wk_ref, wv_ref, q_ref, k_ref, v_ref):
    x = x_ref[...]
    var = jnp.mean(x * x, axis=1, keepdims=True)
    xn = (ln_ref[...] * (x * jax.lax.rsqrt(var + _EPS))).astype(jnp.bfloat16)
    dims = (((1,), (1,)), ((), ()))
    q_ref[...] = jax.lax.dot_general(xn, wq_ref[...].astype(jnp.bfloat16),
                                     dims, preferred_element_type=jnp.float32)
    k_ref[...] = jax.lax.dot_general(xn, wk_ref[...].astype(jnp.bfloat16),
                                     dims, preferred_element_type=jnp.float32)
    v_ref[...] = jax.lax.dot_general(xn, wv_ref[...].astype(jnp.bfloat16),
                                     dims, preferred_element_type=jnp.float32)


def _attn_kernel(q_ref, k_ref, v_ref, cos_ref, sin_ref, o_ref):
    q = q_ref[0]
    k = k_ref[0]
    v = v_ref[0]
    cos = cos_ref[...]
    sin = sin_ref[...]
    half = _DH // 2
    q_rot = jnp.concatenate([-q[:, half:], q[:, :half]], axis=1)
    k_rot = jnp.concatenate([-k[:, half:], k[:, :half]], axis=1)
    qr = ((q * cos + q_rot * sin) * (1.0 / 8.0)).astype(jnp.bfloat16)
    kr = (k * cos + k_rot * sin).astype(jnp.bfloat16)
    scores = jax.lax.dot_general(qr, kr, (((1,), (1,)), ((), ())),
                                 preferred_element_type=jnp.float32)
    m = jnp.max(scores, axis=1, keepdims=True)
    p = jnp.exp(scores - m)
    s = jnp.sum(p, axis=1, keepdims=True)
    o = jnp.dot(p.astype(jnp.bfloat16), v.astype(jnp.bfloat16),
                preferred_element_type=jnp.float32)
    o_ref[0] = o / s


def _post_attn_kernel(a_ref, res_ref, wo_ref, ln_ref, wg_ref,
                      h_ref, hn_ref, comb_ref, zacc_ref):
    a = a_ref[...].astype(jnp.bfloat16)
    h = res_ref[...] + jax.lax.dot_general(
        a, wo_ref[...].astype(jnp.bfloat16), (((1,), (1,)), ((), ())),
        preferred_element_type=jnp.float32)
    h_ref[...] = h
    var = jnp.mean(h * h, axis=1, keepdims=True)
    hn = ln_ref[...] * (h * jax.lax.rsqrt(var + _EPS))
    hn_ref[...] = hn
    logits = jax.lax.dot_general(hn, wg_ref[...], (((1,), (1,)), ((), ())),
                                 preferred_element_type=jnp.float32)
    e = logits.shape[1]
    lm = jnp.max(logits, axis=1, keepdims=True)
    ex = jnp.exp(logits - lm)
    p = ex / jnp.sum(ex, axis=1, keepdims=True)
    iota = jax.lax.broadcasted_iota(jnp.int32, p.shape, 1)
    m1 = jnp.max(p, axis=1, keepdims=True)
    i1 = jnp.min(jnp.where(p == m1, iota, e), axis=1, keepdims=True)
    sel1 = iota == i1
    p2 = jnp.where(sel1, -1.0, p)
    m2 = jnp.max(p2, axis=1, keepdims=True)
    i2 = jnp.min(jnp.where(p2 == m2, iota, e), axis=1, keepdims=True)
    sel2 = iota == i2
    denom = m1 + m2
    comb_ref[...] = (jnp.where(sel1, m1, 0.0) +
                     jnp.where(sel2, m2, 0.0)) / denom
    z = lm + jnp.log(jnp.sum(ex, axis=1, keepdims=True))
    tile_sum = jnp.sum(z * z, axis=0, keepdims=True)

    @pl.when(pl.program_id(0) == 0)
    def _():
        zacc_ref[...] = jnp.zeros_like(zacc_ref)

    zacc_ref[...] += tile_sum


def _moe_kernel(hn_ref, h_ref, comb_ref, wg_ref, wu_ref, wd_ref, o_ref):
    e = pl.program_id(1)
    xs = hn_ref[...].astype(jnp.bfloat16)
    dims = (((1,), (1,)), ((), ()))
    g = jax.lax.dot_general(xs, wg_ref[0].astype(jnp.bfloat16), dims,
                            preferred_element_type=jnp.float32)
    u = jax.lax.dot_general(xs, wu_ref[0].astype(jnp.bfloat16), dims,
                            preferred_element_type=jnp.float32)
    a = (jax.nn.silu(g) * u).astype(jnp.bfloat16)
    y = jax.lax.dot_general(a, wd_ref[0].astype(jnp.bfloat16), dims,
                            preferred_element_type=jnp.float32)
    ne = comb_ref.shape[1]
    onehot = (jax.lax.broadcasted_iota(jnp.int32, (ne, 1), 0) == e
              ).astype(jnp.float32)
    w = jnp.dot(comb_ref[...], onehot, preferred_element_type=jnp.float32)

    @pl.when(e == 0)
    def _():
        o_ref[...] = h_ref[...] + w * y

    @pl.when(e > 0)
    def _():
        o_ref[...] += w * y


@jax.jit
def kernel(hidden_states, ln1_w, ln2_w, Wq, Wk, Wv, Wo, Wg, We_gate, We_up, We_down):
    B, S, D = hidden_states.shape
    E, FF, _ = We_gate.shape
    T = B * S
    TM = 512
    x = hidden_states.reshape(T, D)

    f32 = jnp.float32
    bf16 = jnp.bfloat16
    ln1 = ln1_w.reshape(1, D)
    ln2 = ln2_w.reshape(1, D)

    q, k, v = pl.pallas_call(
        _qkv_kernel,
        grid=(T // TM,),
        in_specs=[
            pl.BlockSpec((TM, D), lambda t: (t, 0)),
            pl.BlockSpec((1, D), lambda t: (0, 0)),
            pl.BlockSpec((D, D), lambda t: (0, 0)),
            pl.BlockSpec((D, D), lambda t: (0, 0)),
            pl.BlockSpec((D, D), lambda t: (0, 0)),
        ],
        out_specs=[
            pl.BlockSpec((TM, D), lambda t: (t, 0)),
            pl.BlockSpec((TM, D), lambda t: (t, 0)),
            pl.BlockSpec((TM, D), lambda t: (t, 0)),
        ],
        out_shape=[jax.ShapeDtypeStruct((T, D), f32)] * 3,
    )(x, ln1, Wq, Wk, Wv)

    def to_heads(t):
        return (t.reshape(B, S, _H, _DH).transpose(0, 2, 1, 3)
                .reshape(B * _H, S, _DH))

    qh, kh, vh = to_heads(q), to_heads(k), to_heads(v)

    inv_freq = 1.0 / (10000.0 ** (jnp.arange(0, _DH, 2, dtype=f32) / _DH))
    t_pos = jnp.arange(S, dtype=f32)
    freqs = jnp.outer(t_pos, inv_freq)
    emb = jnp.concatenate([freqs, freqs], axis=-1)
    cos = jnp.cos(emb)
    sin = jnp.sin(emb)

    attn = pl.pallas_call(
        _attn_kernel,
        grid=(B * _H,),
        in_specs=[
            pl.BlockSpec((1, S, _DH), lambda i: (i, 0, 0)),
            pl.BlockSpec((1, S, _DH), lambda i: (i, 0, 0)),
            pl.BlockSpec((1, S, _DH), lambda i: (i, 0, 0)),
            pl.BlockSpec((S, _DH), lambda i: (0, 0)),
            pl.BlockSpec((S, _DH), lambda i: (0, 0)),
        ],
        out_specs=pl.BlockSpec((1, S, _DH), lambda i: (i, 0, 0)),
        out_shape=jax.ShapeDtypeStruct((B * _H, S, _DH), f32),
    )(qh, kh, vh, cos, sin)

    attn_flat = (attn.reshape(B, _H, S, _DH).transpose(0, 2, 1, 3)
                 .reshape(T, D))

    h, hn, comb, zacc = pl.pallas_call(
        _post_attn_kernel,
        grid=(T // TM,),
        in_specs=[
            pl.BlockSpec((TM, D), lambda t: (t, 0)),
            pl.BlockSpec((TM, D), lambda t: (t, 0)),
            pl.BlockSpec((D, D), lambda t: (0, 0)),
            pl.BlockSpec((1, D), lambda t: (0, 0)),
            pl.BlockSpec((E, D), lambda t: (0, 0)),
        ],
        out_specs=[
            pl.BlockSpec((TM, D), lambda t: (t, 0)),
            pl.BlockSpec((TM, D), lambda t: (t, 0)),
            pl.BlockSpec((TM, E), lambda t: (t, 0)),
            pl.BlockSpec((1, 1), lambda t: (0, 0)),
        ],
        out_shape=[
            jax.ShapeDtypeStruct((T, D), f32),
            jax.ShapeDtypeStruct((T, D), f32),
            jax.ShapeDtypeStruct((T, E), f32),
            jax.ShapeDtypeStruct((1, 1), f32),
        ],
    )(attn_flat, x, Wo, ln2, Wg)

    aux_loss = _Z_LOSS_COEF * zacc[0, 0] / T

    TM2 = 512
    out = pl.pallas_call(
        _moe_kernel,
        grid=(T // TM2, E),
        in_specs=[
            pl.BlockSpec((TM2, D), lambda t, e: (t, 0)),
            pl.BlockSpec((TM2, D), lambda t, e: (t, 0)),
            pl.BlockSpec((TM2, E), lambda t, e: (t, 0)),
            pl.BlockSpec((1, FF, D), lambda t, e: (e, 0, 0)),
            pl.BlockSpec((1, FF, D), lambda t, e: (e, 0, 0)),
            pl.BlockSpec((1, D, FF), lambda t, e: (e, 0, 0)),
        ],
        out_specs=pl.BlockSpec((TM2, D), lambda t, e: (t, 0)),
        out_shape=jax.ShapeDtypeStruct((T, D), f32),
    )(hn, h, comb, We_gate, We_up, We_down)

    return out.reshape(B, S, D), aux_loss
